# baseline (speedup 1.0000x reference)
import functools

import jax
import jax.numpy as jnp
from jax import lax
from jax.experimental import pallas as pl
from jax.experimental.pallas import tpu as pltpu
from jax.experimental.pallas import tpu_sc as plsc

B = 16384
NW = 32
SPW = B // NW
NCHUNK = 2
CS = SPW // NCHUNK


def _sc_body(x_hbm, y_hbm, out_hbm, ybuf, buf, outv, semx, semy):
    wid = lax.axis_index("s") * 2 + lax.axis_index("c")
    base = wid * SPW
    lanes = lax.iota(jnp.int32, 16)

    yh_handles = [
        pltpu.async_copy(y_hbm.at[i, j, pl.ds(base, SPW)],
                         ybuf.at[i * 4 + j], semy)
        for i in range(2) for j in range(4)
    ]

    def start_chunk(c):
        hs = []
        for ch in range(6):
            for r in range(7):
                hs.append(pltpu.async_copy(
                    x_hbm.at[ch, r, :, pl.ds(base + c * CS, CS)],
                    buf.at[pl.ds(ch * 56 + r * 8, 7), :], semx))
        return hs

    handles = start_chunk(0)
    for h in yh_handles:
        h.wait()

    acc_d = jnp.zeros((16,), jnp.float32)
    acc_c = jnp.zeros((16,), jnp.float32)

    for c in range(NCHUNK):
        for h in handles:
            h.wait()

        def dense_body(cell, acc):
            row = (cell // 7) * 8 + lax.rem(cell, 7)
            for k in range(CS // 16):
                a = buf[row, pl.ds(k * 16, 16)]
                acc = acc + a * a
                b = buf[row + 168, pl.ds(k * 16, 16)]
                acc = acc + b * b
            return acc

        acc_d = lax.fori_loop(0, 49, dense_body, acc_d)

        def group_body(g, a_c, c=c):
            col = g * 16 + lanes
            yo = c * CS + g * 16
            r0 = ybuf[0, pl.ds(yo, 16)]
            c0 = ybuf[1, pl.ds(yo, 16)]
            t00 = ybuf[2, pl.ds(yo, 16)]
            t01 = ybuf[3, pl.ds(yo, 16)]
            r1 = ybuf[4, pl.ds(yo, 16)]
            c1 = ybuf[5, pl.ds(yo, 16)]
            t10 = ybuf[6, pl.ds(yo, 16)]
            t11 = ybuf[7, pl.ds(yo, 16)]
            dcell0 = r0.astype(jnp.int32) * 8 + c0.astype(jnp.int32)
            dcell1 = r1.astype(jnp.int32) * 8 + c1.astype(jnp.int32)
            v0 = plsc.load_gather(buf, [dcell0, col])
            g1 = plsc.load_gather(buf, [dcell0 + 56, col])
            g2 = plsc.load_gather(buf, [dcell0 + 112, col])
            v3 = plsc.load_gather(buf, [dcell1 + 168, col])
            g4 = plsc.load_gather(buf, [dcell1 + 224, col])
            g5 = plsc.load_gather(buf, [dcell1 + 280, col])
            corr = ((1.0 - v0) * (1.0 - v0) - 0.5 * v0 * v0
                    + (1.0 - v3) * (1.0 - v3) - 0.5 * v3 * v3)
            d1 = g1 - t00
            d2 = g2 - t01
            d4 = g4 - t10
            d5 = g5 - t11
            det = 5.0 * (d1 * d1 + d2 * d2 + d4 * d4 + d5 * d5)
            return a_c + corr + det

        acc_c = lax.fori_loop(0, CS // 16, group_body, acc_c)

        if c + 1 < NCHUNK:
            handles = start_chunk(c + 1)

    outv[...] = 0.5 * acc_d + acc_c
    pltpu.sync_copy(outv, out_hbm.at[wid])


@jax.jit
def kernel(yh, y):
    x4 = jnp.transpose(yh, (1, 2, 3, 0))
    y3 = jnp.transpose(y, (1, 2, 0))
    mesh = plsc.VectorSubcoreMesh(core_axis_name="c", subcore_axis_name="s")
    partials = pl.kernel(
        _sc_body,
        mesh=mesh,
        compiler_params=pltpu.CompilerParams(needs_layout_passes=False),
        out_type=jax.ShapeDtypeStruct((NW, 16), jnp.float32),
        scratch_types=[
            pltpu.VMEM((8, SPW), jnp.float32),
            pltpu.VMEM((336, CS), jnp.float32),
            pltpu.VMEM((16,), jnp.float32),
            pltpu.SemaphoreType.DMA,
            pltpu.SemaphoreType.DMA,
        ],
    )(x4, y3)
    return jnp.sum(partials)


# speedup vs baseline: 1.1144x; 1.1144x over previous
import functools

import jax
import jax.numpy as jnp
from jax import lax
from jax.experimental import pallas as pl
from jax.experimental.pallas import tpu as pltpu
from jax.experimental.pallas import tpu_sc as plsc

B = 16384
NW = 32
SPW = B // NW
NCHUNK = 4
CS = SPW // NCHUNK
BB = 2048


def _sc_body(x_hbm, y_hbm, out_hbm, ybuf, buf0, buf1, outv, sem0, sem1, semy):
    wid = lax.axis_index("s") * 2 + lax.axis_index("c")
    base = wid * SPW
    lanes = lax.iota(jnp.int32, 16)

    yh_handles = [
        pltpu.async_copy(y_hbm.at[i, j, pl.ds(base, SPW)],
                         ybuf.at[i * 4 + j], semy)
        for i in range(2) for j in range(4)
    ]

    bufs = (buf0, buf1)
    sems = (sem0, sem1)

    def start_chunk(c):
        hs = []
        for i, ch in enumerate((1, 2, 4, 5)):
            for r in range(7):
                hs.append(pltpu.async_copy(
                    x_hbm.at[ch, r, :, pl.ds(base + c * CS, CS)],
                    bufs[c % 2].at[pl.ds((i * 7 + r) * 7, 7), :],
                    sems[c % 2]))
        return hs

    handles = {c: start_chunk(c) for c in range(2)}
    for h in yh_handles:
        h.wait()

    acc_c = jnp.zeros((16,), jnp.float32)

    for c in range(NCHUNK):
        buf = bufs[c % 2]
        for h in handles.pop(c):
            h.wait()

        def group_body(g, a_c, buf=buf, c=c):
            col = g * 16 + lanes
            yo = c * CS + g * 16
            r0 = ybuf[0, pl.ds(yo, 16)]
            c0 = ybuf[1, pl.ds(yo, 16)]
            t00 = ybuf[2, pl.ds(yo, 16)]
            t01 = ybuf[3, pl.ds(yo, 16)]
            r1 = ybuf[4, pl.ds(yo, 16)]
            c1 = ybuf[5, pl.ds(yo, 16)]
            t10 = ybuf[6, pl.ds(yo, 16)]
            t11 = ybuf[7, pl.ds(yo, 16)]
            cell0 = r0.astype(jnp.int32) * 7 + c0.astype(jnp.int32)
            cell1 = r1.astype(jnp.int32) * 7 + c1.astype(jnp.int32)
            g1 = plsc.load_gather(buf, [cell0, col])
            g2 = plsc.load_gather(buf, [cell0 + 49, col])
            g4 = plsc.load_gather(buf, [cell1 + 98, col])
            g5 = plsc.load_gather(buf, [cell1 + 147, col])
            d1 = g1 - t00
            d2 = g2 - t01
            d4 = g4 - t10
            d5 = g5 - t11
            return a_c + 5.0 * (d1 * d1 + d2 * d2 + d4 * d4 + d5 * d5)

        acc_c = lax.fori_loop(0, CS // 16, group_body, acc_c)

        nxt = c + 2
        if nxt < NCHUNK:
            handles[nxt] = start_chunk(nxt)

    outv[...] = acc_c
    pltpu.sync_copy(outv, out_hbm.at[wid])


def _tc_dense(x0_ref, x3_ref, y_ref, out_ref):
    i = pl.program_id(0)

    @pl.when(i == 0)
    def _init():
        out_ref[...] = jnp.zeros((1, 1), jnp.float32)

    a = x0_ref[0]
    b = x3_ref[0]
    r_iota = lax.broadcasted_iota(jnp.int32, (7, 7, BB), 0)
    c_iota = lax.broadcasted_iota(jnp.int32, (7, 7, BB), 1)
    r0 = y_ref[0, 0:1, :].astype(jnp.int32).reshape(1, 1, BB)
    c0 = y_ref[0, 1:2, :].astype(jnp.int32).reshape(1, 1, BB)
    r1 = y_ref[1, 0:1, :].astype(jnp.int32).reshape(1, 1, BB)
    c1 = y_ref[1, 1:2, :].astype(jnp.int32).reshape(1, 1, BB)
    m0 = (r_iota == r0) & (c_iota == c0)
    m3 = (r_iota == r1) & (c_iota == c1)
    term = 0.5 * (a * a + b * b)
    term = term + jnp.where(m0, (1.0 - a) * (1.0 - a) - 0.5 * a * a, 0.0)
    term = term + jnp.where(m3, (1.0 - b) * (1.0 - b) - 0.5 * b * b, 0.0)
    out_ref[...] += jnp.sum(term).reshape(1, 1)


@jax.jit
def kernel(yh, y):
    x4 = jnp.transpose(yh, (1, 2, 3, 0))
    y3 = jnp.transpose(y, (1, 2, 0))
    mesh = plsc.VectorSubcoreMesh(core_axis_name="c", subcore_axis_name="s")
    partials = pl.kernel(
        _sc_body,
        mesh=mesh,
        compiler_params=pltpu.CompilerParams(needs_layout_passes=False),
        out_type=jax.ShapeDtypeStruct((NW, 16), jnp.float32),
        scratch_types=[
            pltpu.VMEM((8, SPW), jnp.float32),
            pltpu.VMEM((196, CS), jnp.float32),
            pltpu.VMEM((196, CS), jnp.float32),
            pltpu.VMEM((16,), jnp.float32),
            pltpu.SemaphoreType.DMA,
            pltpu.SemaphoreType.DMA,
            pltpu.SemaphoreType.DMA,
        ],
    )(x4, y3)
    dense = pl.pallas_call(
        _tc_dense,
        grid=(B // BB,),
        in_specs=[
            pl.BlockSpec((1, 7, 7, BB), lambda i: (0, 0, 0, i)),
            pl.BlockSpec((1, 7, 7, BB), lambda i: (3, 0, 0, i)),
            pl.BlockSpec((2, 4, BB), lambda i: (0, 0, i)),
        ],
        out_specs=pl.BlockSpec((1, 1), lambda i: (0, 0)),
        out_shape=jax.ShapeDtypeStruct((1, 1), jnp.float32),
    )(x4, x4, y3)
    return jnp.sum(partials) + dense[0, 0]


# speedup vs baseline: 1.1556x; 1.0370x over previous
import functools

import jax
import jax.numpy as jnp
from jax import lax
from jax.experimental import pallas as pl
from jax.experimental.pallas import tpu as pltpu
from jax.experimental.pallas import tpu_sc as plsc

B = 16384
NW = 32
SPW = B // NW
NCHUNK = 2
CS = SPW // NCHUNK
BB = 2048


def _sc_body(x_hbm, y_hbm, out_hbm, ybuf, buf0, buf1, outv, sem0, sem1, semy):
    wid = lax.axis_index("s") * 2 + lax.axis_index("c")
    base = wid * SPW
    lanes = lax.iota(jnp.int32, 16)

    yh_handles = [
        pltpu.async_copy(y_hbm.at[i, j, pl.ds(base, SPW)],
                         ybuf.at[i * 4 + j], semy)
        for i in range(2) for j in range(4)
    ]

    bufs = (buf0, buf1)
    sems = (sem0, sem1)

    def start_chunk(c):
        hs = []
        for i, ch in enumerate((1, 2, 4, 5)):
            for r in range(7):
                hs.append(pltpu.async_copy(
                    x_hbm.at[ch, r, :, pl.ds(base + c * CS, CS)],
                    bufs[c % 2].at[pl.ds((i * 7 + r) * 8, 7), :],
                    sems[c % 2]))
        return hs

    handles = {c: start_chunk(c) for c in range(2)}
    for h in yh_handles:
        h.wait()

    acc_c = jnp.zeros((16,), jnp.float32)

    for c in range(NCHUNK):
        buf = bufs[c % 2]
        for h in handles.pop(c):
            h.wait()

        def group_body(g, a_c, buf=buf, c=c):
            col = g * 16 + lanes
            yo = c * CS + g * 16
            r0 = ybuf[0, pl.ds(yo, 16)]
            c0 = ybuf[1, pl.ds(yo, 16)]
            t00 = ybuf[2, pl.ds(yo, 16)]
            t01 = ybuf[3, pl.ds(yo, 16)]
            r1 = ybuf[4, pl.ds(yo, 16)]
            c1 = ybuf[5, pl.ds(yo, 16)]
            t10 = ybuf[6, pl.ds(yo, 16)]
            t11 = ybuf[7, pl.ds(yo, 16)]
            cell0 = r0.astype(jnp.int32) * 8 + c0.astype(jnp.int32)
            cell1 = r1.astype(jnp.int32) * 8 + c1.astype(jnp.int32)
            g1 = plsc.load_gather(buf, [cell0, col])
            g2 = plsc.load_gather(buf, [cell0 + 56, col])
            g4 = plsc.load_gather(buf, [cell1 + 112, col])
            g5 = plsc.load_gather(buf, [cell1 + 168, col])
            d1 = g1 - t00
            d2 = g2 - t01
            d4 = g4 - t10
            d5 = g5 - t11
            return a_c + 5.0 * (d1 * d1 + d2 * d2 + d4 * d4 + d5 * d5)

        acc_c = lax.fori_loop(0, CS // 16, group_body, acc_c)

        nxt = c + 2
        if nxt < NCHUNK:
            handles[nxt] = start_chunk(nxt)

    outv[...] = acc_c
    pltpu.sync_copy(outv, out_hbm.at[wid])


def _tc_dense(x0_ref, x3_ref, y_ref, out_ref):
    i = pl.program_id(0)

    @pl.when(i == 0)
    def _init():
        out_ref[...] = jnp.zeros((1, 1), jnp.float32)

    a = x0_ref[0]
    b = x3_ref[0]
    r_iota = lax.broadcasted_iota(jnp.int32, (7, 7, BB), 0)
    c_iota = lax.broadcasted_iota(jnp.int32, (7, 7, BB), 1)
    r0 = y_ref[0, 0:1, :].astype(jnp.int32).reshape(1, 1, BB)
    c0 = y_ref[0, 1:2, :].astype(jnp.int32).reshape(1, 1, BB)
    r1 = y_ref[1, 0:1, :].astype(jnp.int32).reshape(1, 1, BB)
    c1 = y_ref[1, 1:2, :].astype(jnp.int32).reshape(1, 1, BB)
    m0 = (r_iota == r0) & (c_iota == c0)
    m3 = (r_iota == r1) & (c_iota == c1)
    term = 0.5 * (a * a + b * b)
    term = term + jnp.where(m0, (1.0 - a) * (1.0 - a) - 0.5 * a * a, 0.0)
    term = term + jnp.where(m3, (1.0 - b) * (1.0 - b) - 0.5 * b * b, 0.0)
    out_ref[...] += jnp.sum(term).reshape(1, 1)


@jax.jit
def kernel(yh, y):
    x4 = jnp.transpose(yh, (1, 2, 3, 0))
    y3 = jnp.transpose(y, (1, 2, 0))
    mesh = plsc.VectorSubcoreMesh(core_axis_name="c", subcore_axis_name="s")
    partials = pl.kernel(
        _sc_body,
        mesh=mesh,
        compiler_params=pltpu.CompilerParams(needs_layout_passes=False),
        out_type=jax.ShapeDtypeStruct((NW, 16), jnp.float32),
        scratch_types=[
            pltpu.VMEM((8, SPW), jnp.float32),
            pltpu.VMEM((224, CS), jnp.float32),
            pltpu.VMEM((224, CS), jnp.float32),
            pltpu.VMEM((16,), jnp.float32),
            pltpu.SemaphoreType.DMA,
            pltpu.SemaphoreType.DMA,
            pltpu.SemaphoreType.DMA,
        ],
    )(x4, y3)
    dense = pl.pallas_call(
        _tc_dense,
        grid=(B // BB,),
        in_specs=[
            pl.BlockSpec((1, 7, 7, BB), lambda i: (0, 0, 0, i)),
            pl.BlockSpec((1, 7, 7, BB), lambda i: (3, 0, 0, i)),
            pl.BlockSpec((2, 4, BB), lambda i: (0, 0, i)),
        ],
        out_specs=pl.BlockSpec((1, 1), lambda i: (0, 0)),
        out_shape=jax.ShapeDtypeStruct((1, 1), jnp.float32),
    )(x4, x4, y3)
    return jnp.sum(partials) + dense[0, 0]


# speedup vs baseline: 1.1692x; 1.0118x over previous
import functools

import jax
import jax.numpy as jnp
from jax import lax
from jax.experimental import pallas as pl
from jax.experimental.pallas import tpu as pltpu
from jax.experimental.pallas import tpu_sc as plsc

B = 16384
NW = 32
SPW = B // NW
NCHUNK = 1
CS = SPW // NCHUNK
BB = 2048


def _sc_body(x_hbm, y_hbm, out_hbm, ybuf, buf0, outv, sem0, semy):
    wid = lax.axis_index("s") * 2 + lax.axis_index("c")
    base = wid * SPW
    lanes = lax.iota(jnp.int32, 16)

    yh_handles = [
        pltpu.async_copy(y_hbm.at[i, j, pl.ds(base, SPW)],
                         ybuf.at[i * 4 + j], semy)
        for i in range(2) for j in range(4)
    ]

    bufs = (buf0,)
    sems = (sem0,)

    def start_chunk(c):
        hs = []
        for i, ch in enumerate((1, 2, 4, 5)):
            for r in range(7):
                hs.append(pltpu.async_copy(
                    x_hbm.at[ch, r, :, pl.ds(base + c * CS, CS)],
                    bufs[c % 2].at[pl.ds((i * 7 + r) * 8, 7), :],
                    sems[c % 2]))
        return hs

    handles = {c: start_chunk(c) for c in range(min(2, NCHUNK))}
    for h in yh_handles:
        h.wait()

    acc_c = jnp.zeros((16,), jnp.float32)

    for c in range(NCHUNK):
        buf = bufs[c % 2]
        for h in handles.pop(c):
            h.wait()

        def group_body(g, a_c, buf=buf, c=c):
            col = g * 16 + lanes
            yo = c * CS + g * 16
            r0 = ybuf[0, pl.ds(yo, 16)]
            c0 = ybuf[1, pl.ds(yo, 16)]
            t00 = ybuf[2, pl.ds(yo, 16)]
            t01 = ybuf[3, pl.ds(yo, 16)]
            r1 = ybuf[4, pl.ds(yo, 16)]
            c1 = ybuf[5, pl.ds(yo, 16)]
            t10 = ybuf[6, pl.ds(yo, 16)]
            t11 = ybuf[7, pl.ds(yo, 16)]
            cell0 = r0.astype(jnp.int32) * 8 + c0.astype(jnp.int32)
            cell1 = r1.astype(jnp.int32) * 8 + c1.astype(jnp.int32)
            g1 = plsc.load_gather(buf, [cell0, col])
            g2 = plsc.load_gather(buf, [cell0 + 56, col])
            g4 = plsc.load_gather(buf, [cell1 + 112, col])
            g5 = plsc.load_gather(buf, [cell1 + 168, col])
            d1 = g1 - t00
            d2 = g2 - t01
            d4 = g4 - t10
            d5 = g5 - t11
            return a_c + 5.0 * (d1 * d1 + d2 * d2 + d4 * d4 + d5 * d5)

        acc_c = lax.fori_loop(0, CS // 16, group_body, acc_c)

        nxt = c + 2
        if nxt < NCHUNK:
            handles[nxt] = start_chunk(nxt)

    outv[...] = acc_c
    pltpu.sync_copy(outv, out_hbm.at[wid])


def _tc_dense(x0_ref, x3_ref, y_ref, out_ref):
    i = pl.program_id(0)

    @pl.when(i == 0)
    def _init():
        out_ref[...] = jnp.zeros((1, 1), jnp.float32)

    a = x0_ref[0]
    b = x3_ref[0]
    r_iota = lax.broadcasted_iota(jnp.int32, (7, 7, BB), 0)
    c_iota = lax.broadcasted_iota(jnp.int32, (7, 7, BB), 1)
    r0 = y_ref[0, 0:1, :].astype(jnp.int32).reshape(1, 1, BB)
    c0 = y_ref[0, 1:2, :].astype(jnp.int32).reshape(1, 1, BB)
    r1 = y_ref[1, 0:1, :].astype(jnp.int32).reshape(1, 1, BB)
    c1 = y_ref[1, 1:2, :].astype(jnp.int32).reshape(1, 1, BB)
    m0 = (r_iota == r0) & (c_iota == c0)
    m3 = (r_iota == r1) & (c_iota == c1)
    term = 0.5 * (a * a + b * b)
    term = term + jnp.where(m0, (1.0 - a) * (1.0 - a) - 0.5 * a * a, 0.0)
    term = term + jnp.where(m3, (1.0 - b) * (1.0 - b) - 0.5 * b * b, 0.0)
    out_ref[...] += jnp.sum(term).reshape(1, 1)


@jax.jit
def kernel(yh, y):
    x4 = jnp.transpose(yh, (1, 2, 3, 0))
    y3 = jnp.transpose(y, (1, 2, 0))
    mesh = plsc.VectorSubcoreMesh(core_axis_name="c", subcore_axis_name="s")
    partials = pl.kernel(
        _sc_body,
        mesh=mesh,
        compiler_params=pltpu.CompilerParams(needs_layout_passes=False),
        out_type=jax.ShapeDtypeStruct((NW, 16), jnp.float32),
        scratch_types=[
            pltpu.VMEM((8, SPW), jnp.float32),
            pltpu.VMEM((224, CS), jnp.float32),
            pltpu.VMEM((16,), jnp.float32),
            pltpu.SemaphoreType.DMA,
            pltpu.SemaphoreType.DMA,
        ],
    )(x4, y3)
    dense = pl.pallas_call(
        _tc_dense,
        grid=(B // BB,),
        in_specs=[
            pl.BlockSpec((1, 7, 7, BB), lambda i: (0, 0, 0, i)),
            pl.BlockSpec((1, 7, 7, BB), lambda i: (3, 0, 0, i)),
            pl.BlockSpec((2, 4, BB), lambda i: (0, 0, i)),
        ],
        out_specs=pl.BlockSpec((1, 1), lambda i: (0, 0)),
        out_shape=jax.ShapeDtypeStruct((1, 1), jnp.float32),
    )(x4, x4, y3)
    return jnp.sum(partials) + dense[0, 0]
